# baseline (speedup 1.0000x reference)
import functools

import jax
import jax.numpy as jnp
from jax import lax
from jax.experimental import pallas as pl
from jax.experimental.pallas import tpu as pltpu
from jax.experimental.pallas import tpu_sc as plsc

S = 1024
N = 512
LAT = 27
NCHUNK = 4
CROWS = (2 * S) // NCHUNK

_INV = 1.0 / LAT


def _body(cell_ref, cs_ref, nbr_ref, bg_ref, wg_ref,
          b1_ref, bl_ref, bf2_ref, bd_ref,
          ns_hbm, w1_hbm, wl_hbm, wd_hbm, wf2_hbm,
          out_ref,
          ns_v, w1_v, wl_v, wd_v, wf2_v, sems):
    ns_cp = pltpu.make_async_copy(ns_hbm, ns_v, sems.at[0])
    ns_cp.start()
    w1_cp = pltpu.make_async_copy(w1_hbm, w1_v, sems.at[1])
    w1_cp.start()
    def start_chunks(src, dst, sem_base):
        cps = []
        for c in range(NCHUNK):
            cp = pltpu.make_async_copy(src.at[pl.ds(c * CROWS, CROWS), :],
                                       dst.at[pl.ds(c * CROWS, CROWS), :],
                                       sems.at[sem_base + c])
            cp.start()
            cps.append(cp)
        return cps

    wl_cps = start_chunks(wl_hbm, wl_v, 2)
    wd_cps = start_chunks(wd_hbm, wd_v, 2 + NCHUNK)
    wf2_cps = start_chunks(wf2_hbm, wf2_v, 2 + 2 * NCHUNK)

    idx = nbr_ref[...].astype(jnp.float32)
    q1 = jnp.floor((idx + 0.5) * _INV)
    z = idx - LAT * q1
    q2 = jnp.floor((q1 + 0.5) * _INV)
    y = q1 - LAT * q2
    x = q2
    ci = cell_ref[0, 0].astype(jnp.float32)
    cq1 = jnp.floor((ci + 0.5) * _INV)
    cz = ci - LAT * cq1
    cq2 = jnp.floor((cq1 + 0.5) * _INV)
    cy = cq1 - LAT * cq2
    cx = cq2
    d2 = (x - cx) ** 2 + (y - cy) ** 2 + (z - cz) ** 2
    lm = (d2 < 3.5).astype(jnp.float32)
    dm = (d2 > 24.5).astype(jnp.float32)
    fm = (1.0 - lm) * (1.0 - dm)
    cnt_l = jnp.sum(lm)
    cnt_d = jnp.sum(dm)
    cnt_f = jnp.sum(fm)
    flag_l = (cnt_l > 0.0).astype(jnp.float32)
    flag_d = (cnt_d > 0.0).astype(jnp.float32)
    flag_f = (cnt_f > 0.0).astype(jnp.float32)
    cs = cs_ref[...]

    ns_cp.wait()
    masks = jnp.concatenate(
        [lm, dm, jnp.full((1, N), 1.0 / N, jnp.float32)], axis=0)
    aggs = jnp.dot(masks, ns_v[...], preferred_element_type=jnp.float32)
    agg_l = aggs[0:1] * (1.0 / jnp.maximum(cnt_l, 1.0))
    agg_d = aggs[1:2] * (1.0 / jnp.maximum(cnt_d, 1.0))
    mean_ns = aggs[2:3]
    in_l = jnp.concatenate([cs, agg_l], axis=1)
    in_d = jnp.concatenate([cs, agg_d], axis=1)

    glog = jnp.dot(jnp.concatenate([cs, mean_ns], axis=1), wg_ref[...],
                   preferred_element_type=jnp.float32) + bg_ref[...]
    glog = glog - jnp.max(glog)
    ge = jnp.exp(glog)
    g = ge / jnp.sum(ge)

    w1_cp.wait()
    msg = jnp.tanh(jnp.dot(ns_v[...], w1_v[...],
                           preferred_element_type=jnp.float32) + b1_ref[...])
    agg_f = jnp.dot(fm, msg, preferred_element_type=jnp.float32)
    agg_f = agg_f * (1.0 / jnp.maximum(cnt_f, 1.0))
    in_f = jnp.concatenate([cs, agg_f], axis=1)

    def chunked_matvec(inp, w_v, cps):
        acc = jnp.zeros((1, S), jnp.float32)
        for c in range(NCHUNK):
            cps[c].wait()
            acc = acc + jnp.dot(inp[:, c * CROWS:(c + 1) * CROWS],
                                w_v[c * CROWS:(c + 1) * CROWS, :],
                                preferred_element_type=jnp.float32)
        return acc

    pre_l = chunked_matvec(in_l, wl_v, wl_cps)
    pre_d = chunked_matvec(in_d, wd_v, wd_cps)
    pre_f = chunked_matvec(in_f, wf2_v, wf2_cps)

    lo = jnp.tanh(pre_l + bl_ref[...]) * (g[0, 0] * flag_l)
    fo = jnp.tanh(pre_f + bf2_ref[...]) * (g[0, 1] * flag_f)
    do = jnp.tanh(pre_d + bd_ref[...]) * (g[0, 2] * flag_d)
    out_ref[...] = lo + fo + do


def _mkcall():
    vmem = lambda: pl.BlockSpec(memory_space=pltpu.MemorySpace.VMEM)
    hbm = lambda: pl.BlockSpec(memory_space=pltpu.MemorySpace.HBM)
    return pl.pallas_call(
        _body,
        in_specs=[
            pl.BlockSpec(memory_space=pltpu.MemorySpace.SMEM),
            vmem(),
            vmem(),
            vmem(),
            vmem(),
            vmem(),
            vmem(),
            vmem(),
            vmem(),
            hbm(),
            hbm(),
            hbm(),
            hbm(),
            hbm(),
        ],
        out_specs=pl.BlockSpec(memory_space=pltpu.MemorySpace.VMEM),
        out_shape=jax.ShapeDtypeStruct((1, S), jnp.float32),
        scratch_shapes=[
            pltpu.VMEM((N, S), jnp.float32),
            pltpu.VMEM((S, S), jnp.float32),
            pltpu.VMEM((2 * S, S), jnp.float32),
            pltpu.VMEM((2 * S, S), jnp.float32),
            pltpu.VMEM((2 * S, S), jnp.float32),
            pltpu.SemaphoreType.DMA((2 + 3 * NCHUNK,)),
        ],
    )


L16 = 16
CSLAB = 128
RQ = 128
NCS = S // CSLAB
NRQ = N // RQ
CCH = CSLAB // L16


def _sc_masks(nbr_v, cell_v, m_v, row0):
    cf = cell_v[...].astype(jnp.float32)
    cq1 = ((cf + 0.5) * _INV).astype(jnp.int32).astype(jnp.float32)
    cz = cf - LAT * cq1
    cq2 = ((cq1 + 0.5) * _INV).astype(jnp.int32).astype(jnp.float32)
    cy = cq1 - LAT * cq2
    cx = cq2
    one = jnp.ones((L16,), jnp.float32)
    zero = jnp.zeros((L16,), jnp.float32)

    def mask_step(g, _):
        idx = nbr_v[pl.ds(row0 + g * L16, L16)].astype(jnp.float32)
        q1 = ((idx + 0.5) * _INV).astype(jnp.int32).astype(jnp.float32)
        z = idx - LAT * q1
        q2 = ((q1 + 0.5) * _INV).astype(jnp.int32).astype(jnp.float32)
        y = q1 - LAT * q2
        x = q2
        dx = x - cx
        dy = y - cy
        dz = z - cz
        d2 = dx * dx + dy * dy + dz * dz
        m_v[0, pl.ds(g * L16, L16)] = jnp.where(d2 < 3.5, one, zero)
        m_v[1, pl.ds(g * L16, L16)] = jnp.where(d2 > 24.5, one, zero)
        return 0

    lax.fori_loop(0, RQ // L16, mask_step, 0)


def _sc_body(nbr_hbm, cell_hbm, ns_hbm, out_hbm, nbr_v, cell_v, ns_v, m_v,
             acc_v, sem1):
    wid = lax.axis_index("s") * 2 + lax.axis_index("c")
    cslab = wid % NCS
    rq = wid // NCS
    col0 = cslab * CSLAB
    row0 = rq * RQ
    cp_ns = pltpu.make_async_copy(
        ns_hbm.at[pl.ds(row0, RQ), pl.ds(col0, CSLAB)], ns_v, sem1)
    cp_ns.start()
    pltpu.sync_copy(nbr_hbm, nbr_v)
    pltpu.sync_copy(cell_hbm, cell_v)
    _sc_masks(nbr_v, cell_v, m_v, row0)
    cp_ns.wait()

    zero = jnp.zeros((L16,), jnp.float32)
    init = tuple(zero for _ in range(3 * CCH))

    def group_step(g, carry):
        accs = list(carry)
        base = g * L16
        ml_vec = m_v[0, pl.ds(base, L16)]
        md_vec = m_v[1, pl.ds(base, L16)]
        for j in range(L16):
            ml = ml_vec[j]
            md = md_vec[j]
            for c in range(CCH):
                row = ns_v[base + j, pl.ds(c * L16, L16)]
                accs[c] = accs[c] + ml * row
                accs[CCH + c] = accs[CCH + c] + md * row
                accs[2 * CCH + c] = accs[2 * CCH + c] + row
        return tuple(accs)

    accs = lax.fori_loop(0, RQ // L16, group_step, init)
    for c in range(CCH):
        acc_v[0, pl.ds(c * L16, L16)] = accs[c]
        acc_v[1, pl.ds(c * L16, L16)] = accs[CCH + c]
        acc_v[2, pl.ds(c * L16, L16)] = accs[2 * CCH + c]
        acc_v[3, pl.ds(c * L16, L16)] = zero
    pltpu.sync_copy(acc_v, out_hbm.at[:, pl.ds(rq * S + col0, CSLAB)])


@functools.partial(
    pl.kernel,
    out_type=jax.ShapeDtypeStruct((4, NRQ * S), jnp.float32),
    mesh=plsc.VectorSubcoreMesh(core_axis_name="c", subcore_axis_name="s"),
    scratch_types=[
        pltpu.VMEM((N,), jnp.int32),
        pltpu.VMEM((L16,), jnp.int32),
        pltpu.VMEM((RQ, CSLAB), jnp.float32),
        pltpu.VMEM((2, RQ), jnp.float32),
        pltpu.VMEM((4, CSLAB), jnp.float32),
        pltpu.SemaphoreType.DMA,
    ],
)
def _sc_aggregates(nbr_hbm, cell_hbm, ns_hbm, out_hbm, nbr_v, cell_v, ns_v,
                   m_v, acc_v, sem1):
    _sc_body(nbr_hbm, cell_hbm, ns_hbm, out_hbm, nbr_v, cell_v, ns_v, m_v,
             acc_v, sem1)


def kernel(current_state, neighbor_states, cell_idx, neighbor_indices,
           W_local, b_local, W_func1, b_func1, W_func2, b_func2,
           W_dist, b_dist, W_gate, b_gate):
    cell = jnp.asarray(cell_idx, jnp.int32).reshape(1, 1)
    cs = current_state.reshape(1, S)
    nbr = jnp.asarray(neighbor_indices, jnp.int32).reshape(1, N)
    sc_sums = _sc_aggregates(
        jnp.asarray(neighbor_indices, jnp.int32).reshape(N),
        jnp.full((L16,), jnp.asarray(cell_idx, jnp.int32)),
        neighbor_states,
    )
    out = _mkcall()(
        cell, cs, nbr, b_gate.reshape(1, 3), W_gate,
        b_func1.reshape(1, S), b_local.reshape(1, S),
        b_func2.reshape(1, S), b_dist.reshape(1, S),
        neighbor_states, W_func1, W_local, W_dist, W_func2,
    )
    return out.reshape(S) + jnp.sum(sc_sums[3].reshape(NRQ, S), axis=0)
